# baseline (speedup 1.0000x reference)
import functools

import jax
import jax.numpy as jnp
from jax import lax
from jax.experimental import pallas as pl
from jax.experimental.pallas import tpu as pltpu
from jax.experimental.pallas import tpu_sc as plsc

_NC = 2
_NS = 16
_BN = 2000


def _linear(hs, W, b, relu):
    n = hs[0].shape[0]
    k, o = W.shape
    npc = len(hs)

    def body(*refs):
        h_refs = refs[:npc]
        w_ref, b_ref, o_ref = refs[npc:npc + 3]
        acc = jnp.zeros((_BN, o), jnp.float32)
        off = 0
        for h_ref in h_refs:
            di = h_ref.shape[1]
            acc = acc + jnp.dot(h_ref[...], w_ref[off:off + di, :],
                                preferred_element_type=jnp.float32)
            off += di
        acc = acc + b_ref[...]
        if relu:
            acc = jnp.maximum(acc, 0.0)
        o_ref[...] = acc

    return pl.pallas_call(
        body,
        grid=(n // _BN,),
        in_specs=[pl.BlockSpec((_BN, h.shape[1]), lambda i: (i, 0))
                  for h in hs] +
                 [
            pl.BlockSpec((k, o), lambda i: (0, 0)),
            pl.BlockSpec((1, o), lambda i: (0, 0)),
        ],
        out_specs=pl.BlockSpec((_BN, o), lambda i: (i, 0)),
        out_shape=jax.ShapeDtypeStruct((n, o), jnp.float32),
    )(*hs, W, b.reshape(1, o))


def _rel_matmul(hs, Wr):
    n = hs[0].shape[0]
    r, d, o = Wr.shape
    C = o // 128
    nb = n // _BN
    npc = len(hs)

    def body(*refs):
        h_refs, w_ref, o_ref = refs[:npc], refs[npc], refs[npc + 1]
        m = jnp.zeros((_BN, o), jnp.float32)
        off = 0
        for h_ref in h_refs:
            di = h_ref.shape[1]
            m = m + jnp.dot(h_ref[...], w_ref[0, off:off + di, :],
                            preferred_element_type=jnp.float32)
            off += di
        for c in range(C):
            o_ref[c] = m[:, c * 128:(c + 1) * 128]

    return pl.pallas_call(
        body,
        grid=(r, nb),
        in_specs=[pl.BlockSpec((_BN, h.shape[1]), lambda ri, i: (i, 0))
                  for h in hs] +
                 [pl.BlockSpec((1, d, o), lambda ri, i: (ri, 0, 0))],
        out_specs=pl.BlockSpec((C, _BN, 128), lambda ri, i: (0, ri * nb + i, 0)),
        out_shape=jax.ShapeDtypeStruct((C, r * n, 128), jnp.float32),
    )(*hs, Wr)


def _combine(hs, Ws, b, agg, relu=True):
    n = hs[0].shape[0]
    d, o = Ws.shape
    C = o // 128
    npc = len(hs)

    def body(*refs):
        h_refs = refs[:npc]
        w_ref, b_ref, a_ref, o_ref = refs[npc:npc + 4]
        acc = jnp.zeros((_BN, o), jnp.float32)
        off = 0
        for h_ref in h_refs:
            di = h_ref.shape[1]
            acc = acc + jnp.dot(h_ref[...], w_ref[off:off + di, :],
                                preferred_element_type=jnp.float32)
            off += di
        for c in range(C):
            col = acc[:, c * 128:(c + 1) * 128] + a_ref[c, 0] + a_ref[c, 1]
            col = col + b_ref[0, c * 128:(c + 1) * 128]
            if relu:
                col = jnp.maximum(col, 0.0)
            o_ref[:, c * 128:(c + 1) * 128] = col

    return pl.pallas_call(
        body,
        grid=(n // _BN,),
        in_specs=[pl.BlockSpec((_BN, h.shape[1]), lambda i: (i, 0))
                  for h in hs] +
                 [
            pl.BlockSpec((d, o), lambda i: (0, 0)),
            pl.BlockSpec((1, o), lambda i: (0, 0)),
            pl.BlockSpec((C, 2, _BN, 128), lambda i: (0, 0, i, 0)),
        ],
        out_specs=pl.BlockSpec((_BN, o), lambda i: (i, 0)),
        out_shape=jax.ShapeDtypeStruct((n, o), jnp.float32),
    )(*hs, Ws, b.reshape(1, o), agg)


def _sc_agg(hr_cm, ci3, di3, zrow, n, C, nblk, B):
    rows = (n // _NS) & ~7
    rem = n - rows * _NS
    rem_base = rows * _NS
    zr = 52
    nz = rows // zr
    assert nz * zr == rows and rem <= zr
    mesh = plsc.VectorSubcoreMesh(core_axis_name="c", subcore_axis_name="s")

    @functools.partial(
        pl.kernel,
        out_type=jax.ShapeDtypeStruct((C, _NC, n, 128), jnp.float32),
        mesh=mesh,
        scratch_types=[
            pltpu.VMEM((nblk, B), jnp.int32),
            pltpu.VMEM((nblk, B), jnp.int32),
            pltpu.VMEM((B, 128), jnp.float32),
            pltpu.VMEM((B, 128), jnp.float32),
            pltpu.VMEM((zr, 128), jnp.float32),
            pltpu.VMEM_SHARED((n, 128), jnp.float32),
            pltpu.SemaphoreType.DMA,
            pltpu.SemaphoreType.DMA,
        ],
    )
    def k(hr_ref, ci_ref, di_ref, z_ref, out_ref, cib, dib, gb0, gb1, zb,
          acc, s0, s1):
        core = lax.axis_index("c")
        sub = lax.axis_index("s")
        wid = core * _NS + sub
        pltpu.sync_copy(ci_ref.at[wid], cib)
        pltpu.sync_copy(di_ref.at[wid], dib)
        pltpu.sync_copy(z_ref, zb)
        rbase = sub * rows
        for c in range(C):
            for z in range(nz):
                pltpu.sync_copy(zb, acc.at[pl.ds(rbase + z * zr, zr)])
            if rem:
                @pl.when(sub == 0)
                def _():
                    pltpu.sync_copy(zb.at[pl.ds(0, rem)],
                                    acc.at[pl.ds(rem_base, rem)])
            plsc.subcore_barrier()

            npairs = nblk // 2
            pltpu.async_copy(hr_ref.at[c].at[cib.at[0]], gb0, s0)

            def pair(t, carry):
                j0 = 2 * t
                pltpu.async_copy(hr_ref.at[c].at[cib.at[j0 + 1]], gb1, s1)
                pltpu.make_async_copy(hr_ref.at[c].at[cib.at[j0]], gb0,
                                      s0).wait()
                pltpu.sync_copy(gb0, acc.at[dib.at[j0]], add=True)

                @pl.when(t + 1 < npairs)
                def _():
                    pltpu.async_copy(hr_ref.at[c].at[cib.at[j0 + 2]], gb0, s0)

                pltpu.make_async_copy(hr_ref.at[c].at[cib.at[j0 + 1]], gb1,
                                      s1).wait()
                pltpu.sync_copy(gb1, acc.at[dib.at[j0 + 1]], add=True)
                return carry

            lax.fori_loop(0, npairs, pair, 0)
            plsc.subcore_barrier()
            pltpu.sync_copy(acc.at[pl.ds(rbase, rows)],
                            out_ref.at[c, core, pl.ds(rbase, rows)])
            if rem:
                @pl.when(sub == 0)
                def _():
                    pltpu.sync_copy(acc.at[pl.ds(rem_base, rem)],
                                    out_ref.at[c, core, pl.ds(rem_base, rem)])

    return k(hr_cm, ci3, di3, zrow)


def _gconv(hs, ci3, di3, zrow, Wr, Ws, b, nblk, B):
    n = hs[0].shape[0]
    o = Ws.shape[1]
    C = o // 128
    hr = _rel_matmul(hs, Wr)
    agg = _sc_agg(hr, ci3, di3, zrow, n, C, nblk, B)
    return [_combine(hs, Ws, b, agg)]


def kernel(x, edge_index, edge_type, W_l1, b_l1, Wr_gc1, Ws_gc1, b_gc1,
           Wr_gc2, Ws_gc2, b_gc2, Wr_gc6, Ws_gc6, b_gc6, Wr_gc7, Ws_gc7,
           b_gc7, W_l3, b_l3, Wr_gc3, Ws_gc3, b_gc3, Wr_gc4, Ws_gc4, b_gc4,
           Wr_gc5, Ws_gc5, b_gc5, W_l2, b_l2):
    n = x.shape[0]
    e = edge_index.shape[1]
    src = edge_index[0]
    dst = edge_index[1]
    cidx = edge_type * n + src

    B = 125
    nw = _NC * _NS
    nblk = e // (nw * B)
    ci3 = cidx.reshape(nw, nblk, B)
    di3 = dst.reshape(nw, nblk, B)
    zrow = jnp.zeros((52, 128), jnp.float32)

    kpad = (-x.shape[1]) % 128
    h = [_linear([jnp.pad(x, ((0, 0), (0, kpad)))],
                 jnp.pad(W_l1, ((0, kpad), (0, 0))), b_l1, relu=True)]
    h = _gconv(h, ci3, di3, zrow, Wr_gc1, Ws_gc1, b_gc1, nblk, B)
    h = _gconv(h, ci3, di3, zrow, Wr_gc2, Ws_gc2, b_gc2, nblk, B)
    h = _gconv(h, ci3, di3, zrow, Wr_gc6, Ws_gc6, b_gc6, nblk, B)
    h = _gconv(h, ci3, di3, zrow, Wr_gc7, Ws_gc7, b_gc7, nblk, B)
    h = [_linear(h, W_l3, b_l3, relu=False)]
    h = _gconv(h, ci3, di3, zrow, Wr_gc3, Ws_gc3, b_gc3, nblk, B)
    h = _gconv(h, ci3, di3, zrow, Wr_gc4, Ws_gc4, b_gc4, nblk, B)
    h = _gconv(h, ci3, di3, zrow, Wr_gc5, Ws_gc5, b_gc5, nblk, B)

    opad = (-W_l2.shape[1]) % 128
    out = _linear(h, jnp.pad(W_l2, ((0, 0), (0, opad))),
                  jnp.pad(b_l2, (0, opad)), relu=True)
    return out[:, :W_l2.shape[1]]


# speedup vs baseline: 1.0230x; 1.0230x over previous
import functools

import jax
import jax.numpy as jnp
from jax import lax
from jax.experimental import pallas as pl
from jax.experimental.pallas import tpu as pltpu
from jax.experimental.pallas import tpu_sc as plsc

_NC = 2
_NS = 16
_BN = 2000


def _linear(hs, W, b, relu):
    n = hs[0].shape[0]
    k, o = W.shape
    npc = len(hs)

    def body(*refs):
        h_refs = refs[:npc]
        w_ref, b_ref, o_ref = refs[npc:npc + 3]
        acc = jnp.zeros((_BN, o), jnp.float32)
        off = 0
        for h_ref in h_refs:
            di = h_ref.shape[1]
            acc = acc + jnp.dot(h_ref[...], w_ref[off:off + di, :],
                                preferred_element_type=jnp.float32)
            off += di
        acc = acc + b_ref[...]
        if relu:
            acc = jnp.maximum(acc, 0.0)
        o_ref[...] = acc

    return pl.pallas_call(
        body,
        grid=(n // _BN,),
        in_specs=[pl.BlockSpec((_BN, h.shape[1]), lambda i: (i, 0))
                  for h in hs] +
                 [
            pl.BlockSpec((k, o), lambda i: (0, 0)),
            pl.BlockSpec((1, o), lambda i: (0, 0)),
        ],
        out_specs=pl.BlockSpec((_BN, o), lambda i: (i, 0)),
        out_shape=jax.ShapeDtypeStruct((n, o), jnp.float32),
    )(*hs, W, b.reshape(1, o))


def _rel_matmul(hs, Wr):
    n = hs[0].shape[0]
    r, d, o = Wr.shape
    C = o // 128
    nb = n // _BN
    npc = len(hs)

    def body(*refs):
        h_refs, w_ref, o_ref = refs[:npc], refs[npc], refs[npc + 1]
        m = jnp.zeros((_BN, o), jnp.float32)
        off = 0
        for h_ref in h_refs:
            di = h_ref.shape[1]
            m = m + jnp.dot(h_ref[...], w_ref[0, off:off + di, :],
                            preferred_element_type=jnp.float32)
            off += di
        for c in range(C):
            o_ref[c] = m[:, c * 128:(c + 1) * 128]

    return pl.pallas_call(
        body,
        grid=(nb, r),
        in_specs=[pl.BlockSpec((_BN, h.shape[1]), lambda i, ri: (i, 0))
                  for h in hs] +
                 [pl.BlockSpec((1, d, o), lambda i, ri: (ri, 0, 0))],
        out_specs=pl.BlockSpec((C, _BN, 128), lambda i, ri: (0, ri * nb + i, 0)),
        out_shape=jax.ShapeDtypeStruct((C, r * n, 128), jnp.float32),
    )(*hs, Wr)


def _combine(hs, Ws, b, agg, relu=True):
    n = hs[0].shape[0]
    d, o = Ws.shape
    C = o // 128
    npc = len(hs)

    def body(*refs):
        h_refs = refs[:npc]
        w_ref, b_ref, a_ref, o_ref = refs[npc:npc + 4]
        acc = jnp.zeros((_BN, o), jnp.float32)
        off = 0
        for h_ref in h_refs:
            di = h_ref.shape[1]
            acc = acc + jnp.dot(h_ref[...], w_ref[off:off + di, :],
                                preferred_element_type=jnp.float32)
            off += di
        for c in range(C):
            col = acc[:, c * 128:(c + 1) * 128] + a_ref[c, 0] + a_ref[c, 1]
            col = col + b_ref[0, c * 128:(c + 1) * 128]
            if relu:
                col = jnp.maximum(col, 0.0)
            o_ref[:, c * 128:(c + 1) * 128] = col

    return pl.pallas_call(
        body,
        grid=(n // _BN,),
        in_specs=[pl.BlockSpec((_BN, h.shape[1]), lambda i: (i, 0))
                  for h in hs] +
                 [
            pl.BlockSpec((d, o), lambda i: (0, 0)),
            pl.BlockSpec((1, o), lambda i: (0, 0)),
            pl.BlockSpec((C, 2, _BN, 128), lambda i: (0, 0, i, 0)),
        ],
        out_specs=pl.BlockSpec((_BN, o), lambda i: (i, 0)),
        out_shape=jax.ShapeDtypeStruct((n, o), jnp.float32),
    )(*hs, Ws, b.reshape(1, o), agg)


def _sc_agg(hr_cm, ci3, di3, zrow, n, C, nblk, B):
    rows = (n // _NS) & ~7
    rem = n - rows * _NS
    rem_base = rows * _NS
    zr = 52
    nz = rows // zr
    assert nz * zr == rows and rem <= zr
    mesh = plsc.VectorSubcoreMesh(core_axis_name="c", subcore_axis_name="s")

    @functools.partial(
        pl.kernel,
        out_type=jax.ShapeDtypeStruct((C, _NC, n, 128), jnp.float32),
        mesh=mesh,
        scratch_types=[
            pltpu.VMEM((nblk, B), jnp.int32),
            pltpu.VMEM((nblk, B), jnp.int32),
            pltpu.VMEM((B, 128), jnp.float32),
            pltpu.VMEM((B, 128), jnp.float32),
            pltpu.VMEM((zr, 128), jnp.float32),
            pltpu.VMEM_SHARED((n, 128), jnp.float32),
            pltpu.SemaphoreType.DMA,
            pltpu.SemaphoreType.DMA,
        ],
    )
    def k(hr_ref, ci_ref, di_ref, z_ref, out_ref, cib, dib, gb0, gb1, zb,
          acc, s0, s1):
        core = lax.axis_index("c")
        sub = lax.axis_index("s")
        wid = core * _NS + sub
        pltpu.sync_copy(ci_ref.at[wid], cib)
        pltpu.sync_copy(di_ref.at[wid], dib)
        pltpu.sync_copy(z_ref, zb)
        rbase = sub * rows
        for c in range(C):
            for z in range(nz):
                pltpu.sync_copy(zb, acc.at[pl.ds(rbase + z * zr, zr)])
            if rem:
                @pl.when(sub == 0)
                def _():
                    pltpu.sync_copy(zb.at[pl.ds(0, rem)],
                                    acc.at[pl.ds(rem_base, rem)])
            plsc.subcore_barrier()

            npairs = nblk // 2
            pltpu.async_copy(hr_ref.at[c].at[cib.at[0]], gb0, s0)

            def pair(t, carry):
                j0 = 2 * t
                pltpu.async_copy(hr_ref.at[c].at[cib.at[j0 + 1]], gb1, s1)
                pltpu.make_async_copy(hr_ref.at[c].at[cib.at[j0]], gb0,
                                      s0).wait()
                pltpu.sync_copy(gb0, acc.at[dib.at[j0]], add=True)

                @pl.when(t + 1 < npairs)
                def _():
                    pltpu.async_copy(hr_ref.at[c].at[cib.at[j0 + 2]], gb0, s0)

                pltpu.make_async_copy(hr_ref.at[c].at[cib.at[j0 + 1]], gb1,
                                      s1).wait()
                pltpu.sync_copy(gb1, acc.at[dib.at[j0 + 1]], add=True)
                return carry

            lax.fori_loop(0, npairs, pair, 0)
            plsc.subcore_barrier()
            pltpu.sync_copy(acc.at[pl.ds(rbase, rows)],
                            out_ref.at[c, core, pl.ds(rbase, rows)])
            if rem:
                @pl.when(sub == 0)
                def _():
                    pltpu.sync_copy(acc.at[pl.ds(rem_base, rem)],
                                    out_ref.at[c, core, pl.ds(rem_base, rem)])

    return k(hr_cm, ci3, di3, zrow)


def _gconv(hs, ci3, di3, zrow, Wr, Ws, b, nblk, B):
    n = hs[0].shape[0]
    o = Ws.shape[1]
    C = o // 128
    hr = _rel_matmul(hs, Wr)
    agg = _sc_agg(hr, ci3, di3, zrow, n, C, nblk, B)
    return [_combine(hs, Ws, b, agg)]


def kernel(x, edge_index, edge_type, W_l1, b_l1, Wr_gc1, Ws_gc1, b_gc1,
           Wr_gc2, Ws_gc2, b_gc2, Wr_gc6, Ws_gc6, b_gc6, Wr_gc7, Ws_gc7,
           b_gc7, W_l3, b_l3, Wr_gc3, Ws_gc3, b_gc3, Wr_gc4, Ws_gc4, b_gc4,
           Wr_gc5, Ws_gc5, b_gc5, W_l2, b_l2):
    n = x.shape[0]
    e = edge_index.shape[1]
    src = edge_index[0]
    dst = edge_index[1]
    cidx = edge_type * n + src

    B = 125
    nw = _NC * _NS
    nblk = e // (nw * B)
    ci3 = cidx.reshape(nw, nblk, B)
    di3 = dst.reshape(nw, nblk, B)
    zrow = jnp.zeros((52, 128), jnp.float32)

    kpad = (-x.shape[1]) % 128
    h = [_linear([jnp.pad(x, ((0, 0), (0, kpad)))],
                 jnp.pad(W_l1, ((0, kpad), (0, 0))), b_l1, relu=True)]
    h = _gconv(h, ci3, di3, zrow, Wr_gc1, Ws_gc1, b_gc1, nblk, B)
    h = _gconv(h, ci3, di3, zrow, Wr_gc2, Ws_gc2, b_gc2, nblk, B)
    h = _gconv(h, ci3, di3, zrow, Wr_gc6, Ws_gc6, b_gc6, nblk, B)
    h = _gconv(h, ci3, di3, zrow, Wr_gc7, Ws_gc7, b_gc7, nblk, B)
    h = [_linear(h, W_l3, b_l3, relu=False)]
    h = _gconv(h, ci3, di3, zrow, Wr_gc3, Ws_gc3, b_gc3, nblk, B)
    h = _gconv(h, ci3, di3, zrow, Wr_gc4, Ws_gc4, b_gc4, nblk, B)
    h = _gconv(h, ci3, di3, zrow, Wr_gc5, Ws_gc5, b_gc5, nblk, B)

    opad = (-W_l2.shape[1]) % 128
    out = _linear(h, jnp.pad(W_l2, ((0, 0), (0, opad))),
                  jnp.pad(b_l2, (0, opad)), relu=True)
    return out[:, :W_l2.shape[1]]
